# baseline (speedup 1.0000x reference)
import functools

import jax
import jax.numpy as jnp
from jax import lax
from jax.experimental import pallas as pl
from jax.experimental.pallas import tpu as pltpu
from jax.experimental.pallas import tpu_sc as plsc

_N, _D, _K = 16384, 256, 1024
_BN = 512
_NB = _N // _BN
_NC, _NS = 2, 16
_NW = _NC * _NS
_BPW = _N // _NW
_CH = 64
_NCH = _BPW // _CH


def _rne_bf16(x):
    u = lax.bitcast_convert_type(x, jnp.int32)
    r = (u + jnp.int32(0x7FFF) + ((u >> 16) & jnp.int32(1))) & jnp.int32(-65536)
    return lax.bitcast_convert_type(r, jnp.float32)


def _tc_stage_body(z_ref, w_ref, idx_ref, loss_ref, wb_scr, c2_scr):
    i = pl.program_id(0)

    @pl.when(i == 0)
    def _():
        w = w_ref[...]
        wb_scr[...] = w.astype(jnp.bfloat16)
        c2 = jnp.sum(w * w, axis=1)
        c2_scr[...] = jnp.broadcast_to(c2[None, :], (8, _K))

    r = z_ref[...]
    cl2 = lax.dot_general(
        (r + r).astype(jnp.bfloat16), wb_scr[...],
        (((1,), (1,)), ((), ())), preferred_element_type=jnp.float32)
    l2 = jnp.sum(r * r, axis=1, keepdims=True)
    d = (l2 - cl2) + c2_scr[0:1, :]

    m = jnp.min(d, axis=1, keepdims=True)
    cols = lax.broadcasted_iota(jnp.int32, d.shape, 1)
    idx = jnp.min(jnp.where(d == m, cols, jnp.int32(_K)), axis=1)
    idx_ref[...] = idx.reshape(1, 1, _BN)

    rows8 = lax.broadcasted_iota(jnp.int32, (8, 128), 0)
    cols8 = lax.broadcasted_iota(jnp.int32, (8, 128), 1)
    part = jnp.where((rows8 == 0) & (cols8 == 0), jnp.sum(m), 0.0)

    @pl.when(i == 0)
    def _():
        loss_ref[...] = jnp.zeros_like(loss_ref)

    loss_ref[...] += part


def _tc_stage(r, w):
    row_spec = pl.BlockSpec((_BN, _D), lambda i: (i, 0))
    idx, loss = pl.pallas_call(
        _tc_stage_body,
        grid=(_NB,),
        in_specs=[row_spec, pl.BlockSpec((_K, _D), lambda i: (0, 0))],
        out_specs=[
            pl.BlockSpec((1, 1, _BN), lambda i: (i, 0, 0)),
            pl.BlockSpec((8, 128), lambda i: (0, 0)),
        ],
        out_shape=[
            jax.ShapeDtypeStruct((_NB, 1, _BN), jnp.int32),
            jax.ShapeDtypeStruct((8, 128), jnp.float32),
        ],
        scratch_shapes=[
            pltpu.VMEM((_K, _D), jnp.bfloat16),
            pltpu.VMEM((8, _K), jnp.float32),
        ],
    )(r, w)
    return idx, loss[0, 0]


_sc_mesh = plsc.VectorSubcoreMesh(core_axis_name="c", subcore_axis_name="s")


def _sc_update_chunk(rows_v, rbuf, zbuf):
    def row_body(j, carry):
        for g in range(_D // 16):
            sl = pl.ds(g * 16, 16)
            q = rows_v[j, sl]
            r = rbuf[j, sl]
            qr = _rne_bf16(q)
            c = r + (qr - r)
            rn = r - c
            if zbuf is not None:
                rn = zbuf[j, sl] - rn
            rbuf[j, sl] = rn
        return carry

    lax.fori_loop(0, _CH, row_body, 0)


def _sc_body(final, *refs):
    if final:
        (table_hbm, idx_hbm, rin_hbm, z_hbm, out_hbm,
         idx_v, rows_v, rbuf, zbuf, sem) = refs
    else:
        (table_hbm, idx_hbm, rin_hbm, out_hbm,
         idx_v, rows_v, rbuf, sem) = refs
        z_hbm = zbuf = None
    wid = lax.axis_index("s") * _NC + lax.axis_index("c")
    base = wid * _BPW
    pltpu.sync_copy(idx_hbm.at[wid, 0], idx_v)
    for c in range(_NCH):
        rows = pl.ds(base + c * _CH, _CH)
        gath = pltpu.async_copy(
            table_hbm.at[idx_v.at[pl.ds(c * _CH, _CH)]], rows_v, sem)
        pltpu.sync_copy(rin_hbm.at[rows], rbuf)
        if final:
            pltpu.sync_copy(z_hbm.at[rows], zbuf)
        gath.wait()
        _sc_update_chunk(rows_v, rbuf, zbuf)
        pltpu.sync_copy(rbuf, out_hbm.at[rows])


_sc_residual = pl.kernel(
    functools.partial(_sc_body, False),
    out_type=jax.ShapeDtypeStruct((_N, _D), jnp.float32),
    mesh=_sc_mesh,
    scratch_types=[
        pltpu.VMEM((_BPW,), jnp.int32),
        pltpu.VMEM((_CH, _D), jnp.float32),
        pltpu.VMEM((_CH, _D), jnp.float32),
        pltpu.SemaphoreType.DMA,
    ],
)

_sc_final = pl.kernel(
    functools.partial(_sc_body, True),
    out_type=jax.ShapeDtypeStruct((_N, _D), jnp.float32),
    mesh=_sc_mesh,
    scratch_types=[
        pltpu.VMEM((_BPW,), jnp.int32),
        pltpu.VMEM((_CH, _D), jnp.float32),
        pltpu.VMEM((_CH, _D), jnp.float32),
        pltpu.VMEM((_CH, _D), jnp.float32),
        pltpu.SemaphoreType.DMA,
    ],
)


def kernel(z, codebooks):
    w1 = codebooks[0]
    w2 = codebooks[1]
    w3 = codebooks[2]

    idx1, s1 = _tc_stage(z, w1)
    r2 = _sc_residual(w1, idx1, z)
    idx2, s2 = _tc_stage(r2, w2)
    r3 = _sc_residual(w2, idx2, r2)
    idx3, s3 = _tc_stage(r3, w3)
    fq = _sc_final(w3, idx3, r3, z)

    total = ((s1 + s2) + s3) / jnp.float32(_N * _D)
    return fq, total, total + 0.0


# speedup vs baseline: 1.2686x; 1.2686x over previous
import functools

import jax
import jax.numpy as jnp
from jax import lax
from jax.experimental import pallas as pl
from jax.experimental.pallas import tpu as pltpu
from jax.experimental.pallas import tpu_sc as plsc

_N, _D, _K = 16384, 256, 1024
_BN = 512
_NB = _N // _BN
_NC, _NS = 2, 16
_NW = _NC * _NS
_BPW = _N // _NW
_CH = 64
_NCH = _BPW // _CH


def _rne_bf16(x):
    u = lax.bitcast_convert_type(x, jnp.int32)
    r = (u + jnp.int32(0x7FFF) + ((u >> 16) & jnp.int32(1))) & jnp.int32(-65536)
    return lax.bitcast_convert_type(r, jnp.float32)


def _tc_stage_body(z_ref, w_ref, idx_ref, loss_ref, wb_scr, c2_scr):
    i = pl.program_id(0)

    @pl.when(i == 0)
    def _():
        w = w_ref[...]
        wb_scr[...] = w.astype(jnp.bfloat16)
        c2 = jnp.sum(w * w, axis=1)
        c2_scr[...] = jnp.broadcast_to(c2[None, :], (8, _K))

    r = z_ref[...]
    cl2 = lax.dot_general(
        (r + r).astype(jnp.bfloat16), wb_scr[...],
        (((1,), (1,)), ((), ())), preferred_element_type=jnp.float32)
    l2 = jnp.sum(r * r, axis=1, keepdims=True)
    l2b = jnp.broadcast_to(l2, (_BN, 128))
    c2s = c2_scr[0:1, :]

    m_run = (l2b - cl2[:, 0:128]) + jnp.broadcast_to(c2s[:, 0:128], (_BN, 128))
    i_run = jnp.zeros((_BN, 128), jnp.int32)
    for c in range(1, _K // 128):
        sl = slice(c * 128, (c + 1) * 128)
        dc = (l2b - cl2[:, sl]) + jnp.broadcast_to(c2s[:, sl], (_BN, 128))
        lt = dc < m_run
        m_run = jnp.where(lt, dc, m_run)
        i_run = jnp.where(lt, jnp.int32(c), i_run)

    m = jnp.min(m_run, axis=1, keepdims=True)
    lane = lax.broadcasted_iota(jnp.int32, (_BN, 128), 1)
    kk = i_run * jnp.int32(128) + lane
    idx = jnp.min(jnp.where(m_run == m, kk, jnp.int32(_K)), axis=1)
    idx_ref[...] = idx.reshape(1, 1, _BN)

    rows8 = lax.broadcasted_iota(jnp.int32, (8, 128), 0)
    cols8 = lax.broadcasted_iota(jnp.int32, (8, 128), 1)
    part = jnp.where((rows8 == 0) & (cols8 == 0), jnp.sum(m), 0.0)

    @pl.when(i == 0)
    def _():
        loss_ref[...] = jnp.zeros_like(loss_ref)

    loss_ref[...] += part


def _tc_stage(r, w):
    row_spec = pl.BlockSpec((_BN, _D), lambda i: (i, 0))
    idx, loss = pl.pallas_call(
        _tc_stage_body,
        grid=(_NB,),
        in_specs=[row_spec, pl.BlockSpec((_K, _D), lambda i: (0, 0))],
        out_specs=[
            pl.BlockSpec((1, 1, _BN), lambda i: (i, 0, 0)),
            pl.BlockSpec((8, 128), lambda i: (0, 0)),
        ],
        out_shape=[
            jax.ShapeDtypeStruct((_NB, 1, _BN), jnp.int32),
            jax.ShapeDtypeStruct((8, 128), jnp.float32),
        ],
        scratch_shapes=[
            pltpu.VMEM((_K, _D), jnp.bfloat16),
            pltpu.VMEM((8, _K), jnp.float32),
        ],
    )(r, w)
    return idx, loss[0, 0]


_sc_mesh = plsc.VectorSubcoreMesh(core_axis_name="c", subcore_axis_name="s")


def _sc_update_chunk(rows_v, rbuf, zbuf):
    def row_body(j, carry):
        for g in range(_D // 16):
            sl = pl.ds(g * 16, 16)
            q = rows_v[j, sl]
            r = rbuf[j, sl]
            qr = _rne_bf16(q)
            c = r + (qr - r)
            rn = r - c
            if zbuf is not None:
                rn = zbuf[j, sl] - rn
            rbuf[j, sl] = rn
        return carry

    lax.fori_loop(0, _CH, row_body, 0)


def _sc_body(final, *refs):
    if final:
        (table_hbm, idx_hbm, rin_hbm, z_hbm, out_hbm, idx_v,
         rows0, rows1, rb0, rb1, zb0, zb1,
         sg0, sg1, sr0, sr1, sz0, sz1) = refs
        rows_b, rbuf_b, zbuf_b = (rows0, rows1), (rb0, rb1), (zb0, zb1)
        gsem, rsem, zsem = (sg0, sg1), (sr0, sr1), (sz0, sz1)
    else:
        (table_hbm, idx_hbm, rin_hbm, out_hbm, idx_v,
         rows0, rows1, rb0, rb1,
         sg0, sg1, sr0, sr1) = refs
        rows_b, rbuf_b = (rows0, rows1), (rb0, rb1)
        gsem, rsem = (sg0, sg1), (sr0, sr1)
        z_hbm = zbuf_b = zsem = None
    wid = lax.axis_index("s") * _NC + lax.axis_index("c")
    base = wid * _BPW
    pltpu.sync_copy(idx_hbm.at[wid, 0], idx_v)

    def issue(c):
        b = c % 2
        rows = pl.ds(base + c * _CH, _CH)
        g = pltpu.async_copy(
            table_hbm.at[idx_v.at[pl.ds(c * _CH, _CH)]], rows_b[b], gsem[b])
        r = pltpu.async_copy(rin_hbm.at[rows], rbuf_b[b], rsem[b])
        z = (pltpu.async_copy(z_hbm.at[rows], zbuf_b[b], zsem[b])
             if final else None)
        return g, r, z

    pend = issue(0)
    for c in range(_NCH):
        b = c % 2
        g, r, z = pend
        g.wait()
        r.wait()
        if final:
            z.wait()
        if c + 1 < _NCH:
            pend = issue(c + 1)
        _sc_update_chunk(rows_b[b], rbuf_b[b], zbuf_b[b] if final else None)
        pltpu.sync_copy(rbuf_b[b], out_hbm.at[pl.ds(base + c * _CH, _CH)])


_sc_residual = pl.kernel(
    functools.partial(_sc_body, False),
    out_type=jax.ShapeDtypeStruct((_N, _D), jnp.float32),
    mesh=_sc_mesh,
    scratch_types=(
        [pltpu.VMEM((_BPW,), jnp.int32)]
        + [pltpu.VMEM((_CH, _D), jnp.float32)] * 4
        + [pltpu.SemaphoreType.DMA] * 4
    ),
)

_sc_final = pl.kernel(
    functools.partial(_sc_body, True),
    out_type=jax.ShapeDtypeStruct((_N, _D), jnp.float32),
    mesh=_sc_mesh,
    scratch_types=(
        [pltpu.VMEM((_BPW,), jnp.int32)]
        + [pltpu.VMEM((_CH, _D), jnp.float32)] * 6
        + [pltpu.SemaphoreType.DMA] * 6
    ),
)


def kernel(z, codebooks):
    w1 = codebooks[0]
    w2 = codebooks[1]
    w3 = codebooks[2]

    idx1, s1 = _tc_stage(z, w1)
    r2 = _sc_residual(w1, idx1, z)
    idx2, s2 = _tc_stage(r2, w2)
    r3 = _sc_residual(w2, idx2, r2)
    idx3, s3 = _tc_stage(r3, w3)
    fq = _sc_final(w3, idx3, r3, z)

    total = ((s1 + s2) + s3) / jnp.float32(_N * _D)
    return fq, total, total + 0.0


# speedup vs baseline: 1.4109x; 1.1122x over previous
import functools

import jax
import jax.numpy as jnp
from jax import lax
from jax.experimental import pallas as pl
from jax.experimental.pallas import tpu as pltpu
from jax.experimental.pallas import tpu_sc as plsc

_N, _D, _K = 16384, 256, 1024
_BN = 512
_NB = _N // _BN
_NC, _NS = 2, 16
_NW = _NC * _NS
_BPW = _N // _NW
_CH = 64
_NCH = _BPW // _CH


def _rne_bf16(x):
    u = lax.bitcast_convert_type(x, jnp.int32)
    r = (u + jnp.int32(0x7FFF) + ((u >> 16) & jnp.int32(1))) & jnp.int32(-65536)
    return lax.bitcast_convert_type(r, jnp.float32)


def _tc_stage_body(z_ref, w_ref, idx_ref, loss_ref, wb_scr, c2_scr):
    i = pl.program_id(0)

    @pl.when(i == 0)
    def _():
        w = w_ref[...]
        wb_scr[...] = w.astype(jnp.bfloat16)
        c2 = jnp.sum(w * w, axis=1)
        c2_scr[...] = jnp.broadcast_to(c2[None, :], (8, _K))

    r = z_ref[...]
    cl2 = lax.dot_general(
        (r + r).astype(jnp.bfloat16), wb_scr[...],
        (((1,), (1,)), ((), ())), preferred_element_type=jnp.float32)
    l2 = jnp.sum(r * r, axis=1, keepdims=True)
    l2b = jnp.broadcast_to(l2, (_BN, 128))
    c2s = c2_scr[0:1, :]

    m_run = (l2b - cl2[:, 0:128]) + jnp.broadcast_to(c2s[:, 0:128], (_BN, 128))
    i_run = jnp.zeros((_BN, 128), jnp.int32)
    for c in range(1, _K // 128):
        sl = slice(c * 128, (c + 1) * 128)
        dc = (l2b - cl2[:, sl]) + jnp.broadcast_to(c2s[:, sl], (_BN, 128))
        lt = dc < m_run
        m_run = jnp.where(lt, dc, m_run)
        i_run = jnp.where(lt, jnp.int32(c), i_run)

    m = jnp.min(m_run, axis=1, keepdims=True)
    lane = lax.broadcasted_iota(jnp.int32, (_BN, 128), 1)
    kk = i_run * jnp.int32(128) + lane
    idx = jnp.min(jnp.where(m_run == m, kk, jnp.int32(_K)), axis=1)
    idx_ref[...] = idx.reshape(1, 1, _BN)

    rows8 = lax.broadcasted_iota(jnp.int32, (8, 128), 0)
    cols8 = lax.broadcasted_iota(jnp.int32, (8, 128), 1)
    part = jnp.where((rows8 == 0) & (cols8 == 0), jnp.sum(m), 0.0)

    @pl.when(i == 0)
    def _():
        loss_ref[...] = jnp.zeros_like(loss_ref)

    loss_ref[...] += part


def _tc_stage(r, w, n=_N):
    nb = n // _BN
    row_spec = pl.BlockSpec((_BN, _D), lambda i: (i, 0))
    idx, loss = pl.pallas_call(
        _tc_stage_body,
        grid=(nb,),
        in_specs=[row_spec, pl.BlockSpec((_K, _D), lambda i: (0, 0))],
        out_specs=[
            pl.BlockSpec((1, 1, _BN), lambda i: (i, 0, 0)),
            pl.BlockSpec((8, 128), lambda i: (0, 0)),
        ],
        out_shape=[
            jax.ShapeDtypeStruct((nb, 1, _BN), jnp.int32),
            jax.ShapeDtypeStruct((8, 128), jnp.float32),
        ],
        scratch_shapes=[
            pltpu.VMEM((_K, _D), jnp.bfloat16),
            pltpu.VMEM((8, _K), jnp.float32),
        ],
    )(r, w)
    return idx, loss[0, 0]


_sc_mesh = plsc.VectorSubcoreMesh(core_axis_name="c", subcore_axis_name="s")


def _sc_update_chunk(rows_v, rbuf, zbuf):
    def row_body(j, carry):
        for g in range(_D // 16):
            sl = pl.ds(g * 16, 16)
            q = rows_v[j, sl]
            r = rbuf[j, sl]
            qr = _rne_bf16(q)
            c = r + (qr - r)
            rn = r - c
            if zbuf is not None:
                rn = zbuf[j, sl] - rn
            rbuf[j, sl] = rn
        return carry

    lax.fori_loop(0, _CH, row_body, 0)


def _sc_body(final, bpw, *refs):
    nch = bpw // _CH
    ratio = _BN // bpw
    if final:
        (table_hbm, idx_hbm, rin_hbm, z_hbm, out_hbm, idx_v,
         rows0, rows1, rb0, rb1, zb0, zb1,
         sg0, sg1, sr0, sr1, sz0, sz1) = refs
        rows_b, rbuf_b, zbuf_b = (rows0, rows1), (rb0, rb1), (zb0, zb1)
        gsem, rsem, zsem = (sg0, sg1), (sr0, sr1), (sz0, sz1)
    else:
        (table_hbm, idx_hbm, rin_hbm, out_hbm, idx_v,
         rows0, rows1, rb0, rb1,
         sg0, sg1, sr0, sr1) = refs
        rows_b, rbuf_b = (rows0, rows1), (rb0, rb1)
        gsem, rsem = (sg0, sg1), (sr0, sr1)
        z_hbm = zbuf_b = zsem = None
    wid = lax.axis_index("s") * _NC + lax.axis_index("c")
    base = wid * bpw
    pltpu.sync_copy(
        idx_hbm.at[wid // ratio, 0, pl.ds((wid % ratio) * bpw, bpw)], idx_v)

    def issue(c):
        b = c % 2
        rows = pl.ds(base + c * _CH, _CH)
        g = pltpu.async_copy(
            table_hbm.at[idx_v.at[pl.ds(c * _CH, _CH)]], rows_b[b], gsem[b])
        r = pltpu.async_copy(rin_hbm.at[rows], rbuf_b[b], rsem[b])
        z = (pltpu.async_copy(z_hbm.at[rows], zbuf_b[b], zsem[b])
             if final else None)
        return g, r, z

    pend = issue(0)
    for c in range(nch):
        b = c % 2
        g, r, z = pend
        g.wait()
        r.wait()
        if final:
            z.wait()
        if c + 1 < nch:
            pend = issue(c + 1)
        _sc_update_chunk(rows_b[b], rbuf_b[b], zbuf_b[b] if final else None)
        pltpu.sync_copy(rbuf_b[b], out_hbm.at[pl.ds(base + c * _CH, _CH)])


def _make_sc(final, n):
    bpw = n // _NW
    nbuf = 6 if final else 4
    return pl.kernel(
        functools.partial(_sc_body, final, bpw),
        out_type=jax.ShapeDtypeStruct((n, _D), jnp.float32),
        mesh=_sc_mesh,
        scratch_types=(
            [pltpu.VMEM((bpw,), jnp.int32)]
            + [pltpu.VMEM((_CH, _D), jnp.float32)] * nbuf
            + [pltpu.SemaphoreType.DMA] * nbuf
        ),
    )


_NH = _N // 2
_sc_residual_h = _make_sc(False, _NH)
_sc_final_h = _make_sc(True, _NH)


def _half_chain(zh, w1, w2, w3):
    idx1, s1 = _tc_stage(zh, w1, _NH)
    r2 = _sc_residual_h(w1, idx1, zh)
    idx2, s2 = _tc_stage(r2, w2, _NH)
    r3 = _sc_residual_h(w2, idx2, r2)
    idx3, s3 = _tc_stage(r3, w3, _NH)
    fq = _sc_final_h(w3, idx3, r3, zh)
    return fq, (s1 + s2) + s3


def kernel(z, codebooks):
    w1 = codebooks[0]
    w2 = codebooks[1]
    w3 = codebooks[2]

    za = lax.slice(z, (0, 0), (_NH, _D))
    zb = lax.slice(z, (_NH, 0), (_N, _D))
    fqa, sa = _half_chain(za, w1, w2, w3)
    fqb, sb = _half_chain(zb, w1, w2, w3)
    fq = jnp.concatenate([fqa, fqb], axis=0)

    total = (sa + sb) / jnp.float32(_N * _D)
    return fq, total, total + 0.0
